# baseline (speedup 1.0000x reference)
import functools

import jax
import jax.numpy as jnp
from jax import lax
from jax.experimental import pallas as pl
from jax.experimental.pallas import tpu as pltpu
from jax.experimental.pallas import tpu_sc as plsc

_DIM = 128
_BATCH = 16384
_GAMMA = 12.0
_RHO = 0.5
_BETA = 0.5
_GAMMA_2 = 1.0

_NW = 32
_PER_W = _BATCH // _NW
_G = 64
_NCHUNK = _PER_W // _G
_L = 16


def _vsqrt(x):
    i = lax.bitcast_convert_type(x, jnp.int32)
    i = jnp.int32(0x5F3759DF) - lax.shift_right_logical(i, 1)
    y = lax.bitcast_convert_type(i, jnp.float32)
    y = y * (1.5 - 0.5 * x * y * y)
    y = y * (1.5 - 0.5 * x * y * y)
    y = y * (1.5 - 0.5 * x * y * y)
    return jnp.where(x > 0.0, x * y, 0.0)


def _sc_kernel(rel_hbm, eint_hbm, text_hbm, hidx_hbm, ridx_hbm, tidx_hbm,
               out_hbm,
               hidx_v, ridx_v, tidx_v,
               rel0, hi0, ti0, ht0, tt0,
               rel1, hi1, ti1, ht1, tt1,
               l1p_v, hdp_v, tdp_v,
               out_v,
               isem, sem0, sem1):
    wid = lax.axis_index("s") * 2 + lax.axis_index("c")
    wbase = wid * _PER_W

    wsl = pl.ds(wbase, _PER_W)
    idx_cps = [
        pltpu.async_copy(hidx_hbm.at[wsl], hidx_v, isem),
        pltpu.async_copy(ridx_hbm.at[wsl], ridx_v, isem),
        pltpu.async_copy(tidx_hbm.at[wsl], tidx_v, isem),
    ]
    for cp in idx_cps:
        cp.wait()

    bufs = [(rel0, hi0, ti0, ht0, tt0), (rel1, hi1, ti1, ht1, tt1)]
    sems = [sem0, sem1]

    def fire(c):
        b = c & 1
        o = pl.ds(c * _G, _G)
        rel_v, hi_v, ti_v, ht_v, tt_v = bufs[b]
        return [
            pltpu.async_copy(rel_hbm.at[ridx_v.at[o]], rel_v, sems[b]),
            pltpu.async_copy(eint_hbm.at[hidx_v.at[o]], hi_v, sems[b]),
            pltpu.async_copy(eint_hbm.at[tidx_v.at[o]], ti_v, sems[b]),
            pltpu.async_copy(text_hbm.at[hidx_v.at[o]], ht_v, sems[b]),
            pltpu.async_copy(text_hbm.at[tidx_v.at[o]], tt_v, sems[b]),
        ]

    lane16 = lax.iota(jnp.int32, _L) * _G
    pending = {0: fire(0)}

    for c in range(_NCHUNK):
        if c + 1 < _NCHUNK:
            pending[c + 1] = fire(c + 1)
        for cp in pending.pop(c):
            cp.wait()

        rel_v, hi_v, ti_v, ht_v, tt_v = bufs[c & 1]

        def body(i, carry, rel_v=rel_v, hi_v=hi_v, ti_v=ti_v, ht_v=ht_v,
                 tt_v=tt_v):
            l1 = jnp.zeros((_L,), jnp.float32)
            hd = jnp.zeros((_L,), jnp.float32)
            td = jnp.zeros((_L,), jnp.float32)
            for j in range(_DIM // _L):
                sl = pl.ds(j * _L, _L)
                r = rel_v[i, sl]
                hi = hi_v[i, sl]
                ti = ti_v[i, sl]
                ht = ht_v[i, sl]
                tt = tt_v[i, sl]
                w = 0.5 * ((hi + ht) - (ti + tt)) + r
                l1 = l1 + jnp.abs(w)
                dh = hi - ht
                dt = ti - tt
                hd = hd + dh * dh
                td = td + dt * dt
            tgt = lane16 + i
            plsc.store_scatter(l1p_v, [tgt], l1)
            plsc.store_scatter(hdp_v, [tgt], hd)
            plsc.store_scatter(tdp_v, [tgt], td)
            return carry

        lax.fori_loop(0, _G, body, 0, unroll=2)

        obase = c * _G
        for g in range(_G // _L):
            l1 = l1p_v[pl.ds(g * _L, _L)]
            hd = hdp_v[pl.ds(g * _L, _L)]
            td = tdp_v[pl.ds(g * _L, _L)]
            for l in range(1, _L):
                o = pl.ds(l * _G + g * _L, _L)
                l1 = l1 + l1p_v[o]
                hd = hd + hdp_v[o]
                td = td + tdp_v[o]
            score = (_GAMMA_2 * (_GAMMA - l1)
                     - _BETA * 0.5 * (_vsqrt(hd) + _vsqrt(td)))
            out_v[pl.ds(obase + g * _L, _L)] = score

    pltpu.sync_copy(out_v, out_hbm.at[wsl])


@jax.jit
def kernel(sample, relation_embedding, entity_embedding_init,
           entity_text_embeddings):
    h_idx = sample[:, 0]
    r_idx = sample[:, 1]
    t_idx = sample[:, 2]

    mesh = plsc.VectorSubcoreMesh(core_axis_name="c", subcore_axis_name="s")
    row = pltpu.VMEM((_G, _DIM), jnp.float32)
    part = pltpu.VMEM((_G * _L,), jnp.float32)
    run = functools.partial(
        pl.kernel,
        out_type=jax.ShapeDtypeStruct((_BATCH,), jnp.float32),
        mesh=mesh,
        compiler_params=pltpu.CompilerParams(needs_layout_passes=False),
        scratch_types=[
            pltpu.VMEM((_PER_W,), jnp.int32),
            pltpu.VMEM((_PER_W,), jnp.int32),
            pltpu.VMEM((_PER_W,), jnp.int32),
            row, row, row, row, row,
            row, row, row, row, row,
            part, part, part,
            pltpu.VMEM((_PER_W,), jnp.float32),
            pltpu.SemaphoreType.DMA,
            pltpu.SemaphoreType.DMA,
            pltpu.SemaphoreType.DMA,
        ],
    )(_sc_kernel)
    score = run(relation_embedding, entity_embedding_init,
                entity_text_embeddings, h_idx, r_idx, t_idx)
    return score[:, None]


# speedup vs baseline: 1.2669x; 1.2669x over previous
import functools

import jax
import jax.numpy as jnp
from jax import lax
from jax.experimental import pallas as pl
from jax.experimental.pallas import tpu as pltpu
from jax.experimental.pallas import tpu_sc as plsc

_DIM = 128
_BATCH = 16384
_GAMMA = 12.0
_RHO = 0.5
_BETA = 0.5
_GAMMA_2 = 1.0

_NW = 32
_PER_W = _BATCH // _NW
_G = 64
_NCHUNK = _PER_W // _G
_L = 16


def _vsqrt(x):
    i = lax.bitcast_convert_type(x, jnp.int32)
    i = jnp.int32(0x5F3759DF) - lax.shift_right_logical(i, 1)
    y = lax.bitcast_convert_type(i, jnp.float32)
    y = y * (1.5 - 0.5 * x * y * y)
    y = y * (1.5 - 0.5 * x * y * y)
    y = y * (1.5 - 0.5 * x * y * y)
    return jnp.where(x > 0.0, x * y, 0.0)


def _sc_kernel(rel_hbm, eint_hbm, text_hbm, hidx_hbm, ridx_hbm, tidx_hbm,
               out_hbm,
               hidx_v, ridx_v, tidx_v,
               rel0, hi0, ti0, ht0, tt0,
               rel1, hi1, ti1, ht1, tt1,
               l1p_v, hdp_v, tdp_v,
               out_v,
               isem, sem0, sem1):
    wid = lax.axis_index("s") * 2 + lax.axis_index("c")
    wbase = wid * _PER_W

    wsl = pl.ds(wbase, _PER_W)
    idx_cps = [
        pltpu.async_copy(hidx_hbm.at[wsl], hidx_v, isem),
        pltpu.async_copy(ridx_hbm.at[wsl], ridx_v, isem),
        pltpu.async_copy(tidx_hbm.at[wsl], tidx_v, isem),
    ]
    for cp in idx_cps:
        cp.wait()

    bufs = [(rel0, hi0, ti0, ht0, tt0), (rel1, hi1, ti1, ht1, tt1)]
    sems = [sem0, sem1]

    def fire(c):
        b = c & 1
        o = pl.ds(c * _G, _G)
        rel_v, hi_v, ti_v, ht_v, tt_v = bufs[b]
        return [
            pltpu.async_copy(rel_hbm.at[ridx_v.at[o]], rel_v, sems[b]),
            pltpu.async_copy(eint_hbm.at[hidx_v.at[o]], hi_v, sems[b]),
            pltpu.async_copy(eint_hbm.at[tidx_v.at[o]], ti_v, sems[b]),
            pltpu.async_copy(text_hbm.at[hidx_v.at[o]], ht_v, sems[b]),
            pltpu.async_copy(text_hbm.at[tidx_v.at[o]], tt_v, sems[b]),
        ]

    _S = _G + 1
    lane16 = lax.iota(jnp.int32, _L) * _S
    pending = {0: fire(0)}

    for c in range(_NCHUNK):
        if c + 1 < _NCHUNK:
            pending[c + 1] = fire(c + 1)
        for cp in pending.pop(c):
            cp.wait()

        rel_v, hi_v, ti_v, ht_v, tt_v = bufs[c & 1]

        def body(i, carry, rel_v=rel_v, hi_v=hi_v, ti_v=ti_v, ht_v=ht_v,
                 tt_v=tt_v):
            l1 = jnp.zeros((_L,), jnp.float32)
            hd = jnp.zeros((_L,), jnp.float32)
            td = jnp.zeros((_L,), jnp.float32)
            for j in range(_DIM // _L):
                sl = pl.ds(j * _L, _L)
                r = rel_v[i, sl]
                hi = hi_v[i, sl]
                ti = ti_v[i, sl]
                ht = ht_v[i, sl]
                tt = tt_v[i, sl]
                w = 0.5 * ((hi + ht) - (ti + tt)) + r
                l1 = l1 + jnp.abs(w)
                dh = hi - ht
                dt = ti - tt
                hd = hd + dh * dh
                td = td + dt * dt
            tgt = lane16 + i
            plsc.store_scatter(l1p_v, [tgt], l1)
            plsc.store_scatter(hdp_v, [tgt], hd)
            plsc.store_scatter(tdp_v, [tgt], td)
            return carry

        lax.fori_loop(0, _G, body, 0, unroll=2)

        obase = c * _G
        for g in range(_G // _L):
            l1 = l1p_v[pl.ds(g * _L, _L)]
            hd = hdp_v[pl.ds(g * _L, _L)]
            td = tdp_v[pl.ds(g * _L, _L)]
            for l in range(1, _L):
                o = pl.ds(l * (_G + 1) + g * _L, _L)
                l1 = l1 + l1p_v[o]
                hd = hd + hdp_v[o]
                td = td + tdp_v[o]
            score = (_GAMMA_2 * (_GAMMA - l1)
                     - _BETA * 0.5 * (_vsqrt(hd) + _vsqrt(td)))
            out_v[pl.ds(obase + g * _L, _L)] = score

    pltpu.sync_copy(out_v, out_hbm.at[wsl])


@jax.jit
def kernel(sample, relation_embedding, entity_embedding_init,
           entity_text_embeddings):
    h_idx = sample[:, 0]
    r_idx = sample[:, 1]
    t_idx = sample[:, 2]

    mesh = plsc.VectorSubcoreMesh(core_axis_name="c", subcore_axis_name="s")
    row = pltpu.VMEM((_G, _DIM), jnp.float32)
    part = pltpu.VMEM(((_G + 1) * _L,), jnp.float32)
    run = functools.partial(
        pl.kernel,
        out_type=jax.ShapeDtypeStruct((_BATCH,), jnp.float32),
        mesh=mesh,
        compiler_params=pltpu.CompilerParams(needs_layout_passes=False),
        scratch_types=[
            pltpu.VMEM((_PER_W,), jnp.int32),
            pltpu.VMEM((_PER_W,), jnp.int32),
            pltpu.VMEM((_PER_W,), jnp.int32),
            row, row, row, row, row,
            row, row, row, row, row,
            part, part, part,
            pltpu.VMEM((_PER_W,), jnp.float32),
            pltpu.SemaphoreType.DMA,
            pltpu.SemaphoreType.DMA,
            pltpu.SemaphoreType.DMA,
        ],
    )(_sc_kernel)
    score = run(relation_embedding, entity_embedding_init,
                entity_text_embeddings, h_idx, r_idx, t_idx)
    return score[:, None]


# speedup vs baseline: 1.3329x; 1.0521x over previous
import functools

import jax
import jax.numpy as jnp
from jax import lax
from jax.experimental import pallas as pl
from jax.experimental.pallas import tpu as pltpu
from jax.experimental.pallas import tpu_sc as plsc

_DIM = 128
_BATCH = 16384
_GAMMA = 12.0
_RHO = 0.5
_BETA = 0.5
_GAMMA_2 = 1.0

_NW = 32
_PER_W = _BATCH // _NW
_G = 64
_NCHUNK = _PER_W // _G
_L = 16


def _vsqrt(x):
    i = lax.bitcast_convert_type(x, jnp.int32)
    i = jnp.int32(0x5F3759DF) - lax.shift_right_logical(i, 1)
    y = lax.bitcast_convert_type(i, jnp.float32)
    y = y * (1.5 - 0.5 * x * y * y)
    y = y * (1.5 - 0.5 * x * y * y)
    y = y * (1.5 - 0.5 * x * y * y)
    return jnp.where(x > 0.0, x * y, 0.0)


def _sc_kernel(rel_hbm, eint_hbm, text_hbm, hidx_hbm, ridx_hbm, tidx_hbm,
               out_hbm,
               hidx_v, ridx_v, tidx_v,
               rel0, hi0, ti0, ht0, tt0,
               rel1, hi1, ti1, ht1, tt1,
               out_v,
               isem, sem0, sem1):
    wid = lax.axis_index("s") * 2 + lax.axis_index("c")
    wbase = wid * _PER_W

    wsl = pl.ds(wbase, _PER_W)
    idx_cps = [
        pltpu.async_copy(hidx_hbm.at[wsl], hidx_v, isem),
        pltpu.async_copy(ridx_hbm.at[wsl], ridx_v, isem),
        pltpu.async_copy(tidx_hbm.at[wsl], tidx_v, isem),
    ]
    for cp in idx_cps:
        cp.wait()

    bufs = [(rel0, hi0, ti0, ht0, tt0), (rel1, hi1, ti1, ht1, tt1)]
    sems = [sem0, sem1]
    iota = lax.iota(jnp.int32, _L)

    def descs(c, b):
        o = pl.ds(c * _G, _G)
        rel_v, hi_v, ti_v, ht_v, tt_v = bufs[b]
        return [
            pltpu.make_async_copy(rel_hbm.at[ridx_v.at[o]], rel_v, sems[b]),
            pltpu.make_async_copy(eint_hbm.at[hidx_v.at[o]], hi_v, sems[b]),
            pltpu.make_async_copy(eint_hbm.at[tidx_v.at[o]], ti_v, sems[b]),
            pltpu.make_async_copy(text_hbm.at[hidx_v.at[o]], ht_v, sems[b]),
            pltpu.make_async_copy(text_hbm.at[tidx_v.at[o]], tt_v, sems[b]),
        ]

    def fire(c, b):
        for cp in descs(c, b):
            cp.start()

    def drain(c, b):
        for cp in descs(c, b):
            cp.wait()

    def compute(c, b):
        rel_v, hi_v, ti_v, ht_v, tt_v = bufs[b]
        obase = c * _G

        def body(i, carry):
            l1acc, hdacc, tdacc = carry
            l1 = jnp.zeros((_L,), jnp.float32)
            hd = jnp.zeros((_L,), jnp.float32)
            td = jnp.zeros((_L,), jnp.float32)
            for j in range(_DIM // _L):
                sl = pl.ds(j * _L, _L)
                r = rel_v[i, sl]
                hi = hi_v[i, sl]
                ti = ti_v[i, sl]
                ht = ht_v[i, sl]
                tt = tt_v[i, sl]
                w = 0.5 * ((hi + ht) - (ti + tt)) + r
                l1 = l1 + jnp.abs(w)
                dh = hi - ht
                dt = ti - tt
                hd = hd + dh * dh
                td = td + dt * dt
            m = iota == (i & (_L - 1))
            l1acc = jnp.where(m, jnp.sum(l1), l1acc)
            hdacc = jnp.where(m, jnp.sum(hd), hdacc)
            tdacc = jnp.where(m, jnp.sum(td), tdacc)

            @pl.when((i & (_L - 1)) == (_L - 1))
            def _():
                score = (_GAMMA_2 * (_GAMMA - l1acc)
                         - _BETA * 0.5 * (_vsqrt(hdacc) + _vsqrt(tdacc)))
                out_v[pl.ds(obase + i - (_L - 1), _L)] = score

            return (l1acc, hdacc, tdacc)

        zeros = jnp.zeros((_L,), jnp.float32)
        lax.fori_loop(0, _G, body, (zeros, zeros, zeros), unroll=2)

    fire(0, 0)
    fire(1, 1)

    def pair(p, carry):
        c0 = 2 * p
        drain(c0, 0)
        compute(c0, 0)

        @pl.when(p < _NCHUNK // 2 - 1)
        def _():
            fire(c0 + 2, 0)

        drain(c0 + 1, 1)
        compute(c0 + 1, 1)

        @pl.when(p < _NCHUNK // 2 - 1)
        def _():
            fire(c0 + 3, 1)

        return carry

    lax.fori_loop(0, _NCHUNK // 2, pair, 0)

    pltpu.sync_copy(out_v, out_hbm.at[wsl])


@jax.jit
def kernel(sample, relation_embedding, entity_embedding_init,
           entity_text_embeddings):
    h_idx = sample[:, 0]
    r_idx = sample[:, 1]
    t_idx = sample[:, 2]

    mesh = plsc.VectorSubcoreMesh(core_axis_name="c", subcore_axis_name="s")
    row = pltpu.VMEM((_G, _DIM), jnp.float32)
    run = functools.partial(
        pl.kernel,
        out_type=jax.ShapeDtypeStruct((_BATCH,), jnp.float32),
        mesh=mesh,
        compiler_params=pltpu.CompilerParams(needs_layout_passes=False),
        scratch_types=[
            pltpu.VMEM((_PER_W,), jnp.int32),
            pltpu.VMEM((_PER_W,), jnp.int32),
            pltpu.VMEM((_PER_W,), jnp.int32),
            row, row, row, row, row,
            row, row, row, row, row,
            pltpu.VMEM((_PER_W,), jnp.float32),
            pltpu.SemaphoreType.DMA,
            pltpu.SemaphoreType.DMA,
            pltpu.SemaphoreType.DMA,
        ],
    )(_sc_kernel)
    score = run(relation_embedding, entity_embedding_init,
                entity_text_embeddings, h_idx, r_idx, t_idx)
    return score[:, None]


# speedup vs baseline: 1.3955x; 1.0470x over previous
import functools

import jax
import jax.numpy as jnp
from jax import lax
from jax.experimental import pallas as pl
from jax.experimental.pallas import tpu as pltpu
from jax.experimental.pallas import tpu_sc as plsc

_DIM = 128
_BATCH = 16384
_GAMMA = 12.0
_RHO = 0.5
_BETA = 0.5
_GAMMA_2 = 1.0

_NW = 32
_PER_W = _BATCH // _NW
_G = 64
_NCHUNK = _PER_W // _G
_L = 16


def _vsqrt(x):
    i = lax.bitcast_convert_type(x, jnp.int32)
    i = jnp.int32(0x5F3759DF) - lax.shift_right_logical(i, 1)
    y = lax.bitcast_convert_type(i, jnp.float32)
    y = y * (1.5 - 0.5 * x * y * y)
    y = y * (1.5 - 0.5 * x * y * y)
    y = y * (1.5 - 0.5 * x * y * y)
    return jnp.where(x > 0.0, x * y, 0.0)


def _sc_kernel(rel_hbm, eint_hbm, text_hbm, hidx_hbm, ridx_hbm, tidx_hbm,
               out_hbm,
               hidx_v, ridx_v, tidx_v,
               rel0, hi0, ti0, ht0, tt0,
               rel1, hi1, ti1, ht1, tt1,
               out_v,
               isem, sem0, sem1):
    wid = lax.axis_index("s") * 2 + lax.axis_index("c")
    wbase = wid * _PER_W

    wsl = pl.ds(wbase, _PER_W)
    idx_cps = [
        pltpu.async_copy(hidx_hbm.at[wsl], hidx_v, isem),
        pltpu.async_copy(ridx_hbm.at[wsl], ridx_v, isem),
        pltpu.async_copy(tidx_hbm.at[wsl], tidx_v, isem),
    ]
    for cp in idx_cps:
        cp.wait()

    bufs = [(rel0, hi0, ti0, ht0, tt0), (rel1, hi1, ti1, ht1, tt1)]
    sems = [sem0, sem1]
    iota = lax.iota(jnp.int32, _L)

    def descs(c, b):
        o = pl.ds(c * _G, _G)
        rel_v, hi_v, ti_v, ht_v, tt_v = bufs[b]
        return [
            pltpu.make_async_copy(rel_hbm.at[ridx_v.at[o]], rel_v, sems[b]),
            pltpu.make_async_copy(eint_hbm.at[hidx_v.at[o]], hi_v, sems[b]),
            pltpu.make_async_copy(eint_hbm.at[tidx_v.at[o]], ti_v, sems[b]),
            pltpu.make_async_copy(text_hbm.at[hidx_v.at[o]], ht_v, sems[b]),
            pltpu.make_async_copy(text_hbm.at[tidx_v.at[o]], tt_v, sems[b]),
        ]

    def fire(c, b):
        for cp in descs(c, b):
            cp.start()

    def drain(c, b):
        for cp in descs(c, b):
            cp.wait()

    def compute(c, b):
        rel_v, hi_v, ti_v, ht_v, tt_v = bufs[b]
        obase = c * _G

        def body(i, carry):
            l1acc, hdacc, tdacc = carry
            l1 = jnp.zeros((_L,), jnp.float32)
            hd = jnp.zeros((_L,), jnp.float32)
            td = jnp.zeros((_L,), jnp.float32)
            for j in range(_DIM // _L):
                sl = pl.ds(j * _L, _L)
                r = rel_v[i, sl]
                hi = hi_v[i, sl]
                ti = ti_v[i, sl]
                ht = ht_v[i, sl]
                tt = tt_v[i, sl]
                w = 0.5 * ((hi + ht) - (ti + tt)) + r
                l1 = l1 + jnp.abs(w)
                dh = hi - ht
                dt = ti - tt
                hd = hd + dh * dh
                td = td + dt * dt
            m = iota == (i & (_L - 1))
            l1acc = jnp.where(m, jnp.sum(l1), l1acc)
            hdacc = jnp.where(m, jnp.sum(hd), hdacc)
            tdacc = jnp.where(m, jnp.sum(td), tdacc)

            @pl.when((i & (_L - 1)) == (_L - 1))
            def _():
                score = (_GAMMA_2 * (_GAMMA - l1acc)
                         - _BETA * 0.5 * (_vsqrt(hdacc) + _vsqrt(tdacc)))
                out_v[pl.ds(obase + i - (_L - 1), _L)] = score

            return (l1acc, hdacc, tdacc)

        zeros = jnp.zeros((_L,), jnp.float32)
        lax.fori_loop(0, _G, body, (zeros, zeros, zeros), unroll=4)

    fire(0, 0)
    fire(1, 1)

    def pair(p, carry):
        c0 = 2 * p
        drain(c0, 0)
        compute(c0, 0)

        @pl.when(p < _NCHUNK // 2 - 1)
        def _():
            fire(c0 + 2, 0)

        drain(c0 + 1, 1)
        compute(c0 + 1, 1)

        @pl.when(p < _NCHUNK // 2 - 1)
        def _():
            fire(c0 + 3, 1)

        return carry

    lax.fori_loop(0, _NCHUNK // 2, pair, 0)

    pltpu.sync_copy(out_v, out_hbm.at[wsl])


@jax.jit
def kernel(sample, relation_embedding, entity_embedding_init,
           entity_text_embeddings):
    h_idx = sample[:, 0]
    r_idx = sample[:, 1]
    t_idx = sample[:, 2]

    mesh = plsc.VectorSubcoreMesh(core_axis_name="c", subcore_axis_name="s")
    row = pltpu.VMEM((_G, _DIM), jnp.float32)
    run = functools.partial(
        pl.kernel,
        out_type=jax.ShapeDtypeStruct((_BATCH,), jnp.float32),
        mesh=mesh,
        compiler_params=pltpu.CompilerParams(needs_layout_passes=False),
        scratch_types=[
            pltpu.VMEM((_PER_W,), jnp.int32),
            pltpu.VMEM((_PER_W,), jnp.int32),
            pltpu.VMEM((_PER_W,), jnp.int32),
            row, row, row, row, row,
            row, row, row, row, row,
            pltpu.VMEM((_PER_W,), jnp.float32),
            pltpu.SemaphoreType.DMA,
            pltpu.SemaphoreType.DMA,
            pltpu.SemaphoreType.DMA,
        ],
    )(_sc_kernel)
    score = run(relation_embedding, entity_embedding_init,
                entity_text_embeddings, h_idx, r_idx, t_idx)
    return score[:, None]
